# Initial kernel scaffold; baseline (speedup 1.0000x reference)
import jax
import jax.numpy as jnp
from jax.experimental import pallas as pl


def kernel(word, pos1, pos2, word_table, pos1_table, pos2_table):
    raise NotImplementedError("write your pallas kernel here")


# speedup vs baseline: 3.6000x; 3.6000x over previous
import functools

import jax
import jax.numpy as jnp
from jax import lax
from jax.experimental import pallas as pl
from jax.experimental.pallas import tpu as pltpu
from jax.experimental.pallas import tpu_sc as plsc

B = 4096
L = 200
N = B * L
WORD_DIM = 64
POS_DIM = 16
OUT_DIM = 96

NC = 2
NS = 16
NW = NC * NS
PER_W = N // NW
CHUNK = 256
NIT = PER_W // CHUNK


def _emb_body(word_hbm, pos1_hbm, pos2_hbm, wt_hbm, p1t_hbm, p2t_hbm,
              out_hbm, widx_s, p1idx_s, p2idx_s, p1t_v, p2t_v, out_v, sem):
    c = lax.axis_index("c")
    s = lax.axis_index("s")
    wid = s * NC + c
    base0 = wid * PER_W

    pltpu.sync_copy(p1t_hbm, p1t_v)
    pltpu.sync_copy(p2t_hbm, p2t_v)

    def body(it, carry):
        base = base0 + it * CHUNK
        pltpu.sync_copy(word_hbm.at[pl.ds(base, CHUNK)], widx_s)
        pltpu.sync_copy(pos1_hbm.at[pl.ds(base, CHUNK)], p1idx_s)
        pltpu.sync_copy(pos2_hbm.at[pl.ds(base, CHUNK)], p2idx_s)

        def group(g, carry2):
            r0 = g * 16
            wv = widx_s[pl.ds(r0, 16)]
            p1v = p1idx_s[pl.ds(r0, 16)]
            p2v = p2idx_s[pl.ds(r0, 16)]
            for u in range(16):
                r = r0 + u
                pltpu.async_copy(
                    wt_hbm.at[wv[u]], out_v.at[r, pl.ds(0, WORD_DIM)], sem)
                out_v[r, pl.ds(WORD_DIM, POS_DIM)] = p1t_v[p1v[u], :]
                out_v[r, pl.ds(WORD_DIM + POS_DIM, POS_DIM)] = p2t_v[p2v[u], :]
            return carry2

        lax.fori_loop(0, CHUNK // 16, group, 0)
        pltpu.make_async_copy(
            wt_hbm.at[pl.ds(0, CHUNK)], out_v.at[:, pl.ds(0, WORD_DIM)], sem
        ).wait()
        pltpu.sync_copy(out_v, out_hbm.at[pl.ds(base, CHUNK)])
        return carry

    lax.fori_loop(0, NIT, body, 0)


@jax.jit
def _emb(word_f, pos1_f, pos2_f, word_table, pos1_table, pos2_table):
    mesh = plsc.VectorSubcoreMesh(core_axis_name="c", subcore_axis_name="s")
    f = pl.kernel(
        _emb_body,
        mesh=mesh,
        compiler_params=pltpu.CompilerParams(use_tc_tiling_on_sc=False),
        out_type=jax.ShapeDtypeStruct((N, OUT_DIM), jnp.float32),
        scratch_types=[
            pltpu.VMEM((CHUNK,), jnp.int32),
            pltpu.VMEM((CHUNK,), jnp.int32),
            pltpu.VMEM((CHUNK,), jnp.int32),
            pltpu.VMEM((512, POS_DIM), jnp.float32),
            pltpu.VMEM((512, POS_DIM), jnp.float32),
            pltpu.VMEM((CHUNK, OUT_DIM), jnp.float32),
            pltpu.SemaphoreType.DMA,
        ],
    )
    return f(word_f, pos1_f, pos2_f, word_table, pos1_table, pos2_table)


def kernel(word, pos1, pos2, word_table, pos1_table, pos2_table):
    word_f = word.astype(jnp.int32).reshape(N)
    pos1_f = pos1.astype(jnp.int32).reshape(N)
    pos2_f = pos2.astype(jnp.int32).reshape(N)
    out = _emb(word_f, pos1_f, pos2_f, word_table, pos1_table, pos2_table)
    return out.reshape(B, L, OUT_DIM)


# speedup vs baseline: 4.1838x; 1.1622x over previous
import functools

import jax
import jax.numpy as jnp
from jax import lax
from jax.experimental import pallas as pl
from jax.experimental.pallas import tpu as pltpu
from jax.experimental.pallas import tpu_sc as plsc

B = 4096
L = 200
N = B * L
WORD_DIM = 64
POS_DIM = 16
OUT_DIM = 96

NC = 2
NS = 16
NW = NC * NS
PER_W = N // NW
CHUNK = 512
NIT = PER_W // CHUNK


def _emb_body(word_hbm, pos1_hbm, pos2_hbm, wt_hbm, p1t_hbm, p2t_hbm,
              out_hbm, widx, p1idx, p2idx, p1t_v, p2t_v, ob,
              gsem, isem, wsem):
    c = lax.axis_index("c")
    s = lax.axis_index("s")
    wid = s * NC + c
    base0 = wid * PER_W

    pltpu.sync_copy(p1t_hbm, p1t_v)
    pltpu.sync_copy(p2t_hbm, p2t_v)

    pltpu.sync_copy(word_hbm.at[pl.ds(base0, CHUNK)], widx.at[pl.ds(0, CHUNK)])
    pltpu.sync_copy(pos1_hbm.at[pl.ds(base0, CHUNK)], p1idx.at[pl.ds(0, CHUNK)])
    pltpu.sync_copy(pos2_hbm.at[pl.ds(base0, CHUNK)], p2idx.at[pl.ds(0, CHUNK)])

    def body(it, carry):
        slot = lax.rem(it, 2)
        nslot = 1 - slot
        base = base0 + it * CHUNK
        soff = slot * CHUNK

        @pl.when(it + 1 < NIT)
        def _():
            nbase = base + CHUNK
            noff = nslot * CHUNK
            pltpu.async_copy(word_hbm.at[pl.ds(nbase, CHUNK)],
                             widx.at[pl.ds(noff, CHUNK)], isem)
            pltpu.async_copy(pos1_hbm.at[pl.ds(nbase, CHUNK)],
                             p1idx.at[pl.ds(noff, CHUNK)], isem)
            pltpu.async_copy(pos2_hbm.at[pl.ds(nbase, CHUNK)],
                             p2idx.at[pl.ds(noff, CHUNK)], isem)

        @pl.when(it >= 2)
        def _():
            pltpu.make_async_copy(
                ob.at[pl.ds(soff, CHUNK)],
                out_hbm.at[pl.ds(0, CHUNK)], wsem).wait()

        def group(g, carry2):
            r0 = soff + g * 16
            wv = widx[pl.ds(r0, 16)]
            p1v = p1idx[pl.ds(r0, 16)]
            p2v = p2idx[pl.ds(r0, 16)]
            for u in range(16):
                r = r0 + u
                pltpu.async_copy(
                    wt_hbm.at[wv[u]], ob.at[r, pl.ds(0, WORD_DIM)], gsem)
                ob[r, pl.ds(WORD_DIM, POS_DIM)] = p1t_v[p1v[u], :]
                ob[r, pl.ds(WORD_DIM + POS_DIM, POS_DIM)] = p2t_v[p2v[u], :]
            return carry2

        lax.fori_loop(0, CHUNK // 16, group, 0)

        pltpu.make_async_copy(
            wt_hbm.at[pl.ds(0, CHUNK)],
            ob.at[pl.ds(soff, CHUNK), pl.ds(0, WORD_DIM)], gsem).wait()

        @pl.when(it + 1 < NIT)
        def _():
            for ref in (widx, p1idx, p2idx):
                pltpu.make_async_copy(
                    word_hbm.at[pl.ds(0, CHUNK)],
                    ref.at[pl.ds(0, CHUNK)], isem).wait()

        pltpu.async_copy(ob.at[pl.ds(soff, CHUNK)],
                         out_hbm.at[pl.ds(base, CHUNK)], wsem)
        return carry

    lax.fori_loop(0, NIT, body, 0)

    for _ in range(2):
        pltpu.make_async_copy(
            ob.at[pl.ds(0, CHUNK)], out_hbm.at[pl.ds(0, CHUNK)], wsem).wait()


@jax.jit
def _emb(word_f, pos1_f, pos2_f, word_table, pos1_table, pos2_table):
    mesh = plsc.VectorSubcoreMesh(core_axis_name="c", subcore_axis_name="s")
    f = pl.kernel(
        _emb_body,
        mesh=mesh,
        compiler_params=pltpu.CompilerParams(use_tc_tiling_on_sc=False),
        out_type=jax.ShapeDtypeStruct((N, OUT_DIM), jnp.float32),
        scratch_types=[
            pltpu.VMEM((2 * CHUNK,), jnp.int32),
            pltpu.VMEM((2 * CHUNK,), jnp.int32),
            pltpu.VMEM((2 * CHUNK,), jnp.int32),
            pltpu.VMEM((512, POS_DIM), jnp.float32),
            pltpu.VMEM((512, POS_DIM), jnp.float32),
            pltpu.VMEM((2 * CHUNK, OUT_DIM), jnp.float32),
            pltpu.SemaphoreType.DMA,
            pltpu.SemaphoreType.DMA,
            pltpu.SemaphoreType.DMA,
        ],
    )
    return f(word_f, pos1_f, pos2_f, word_table, pos1_table, pos2_table)


def kernel(word, pos1, pos2, word_table, pos1_table, pos2_table):
    word_f = word.astype(jnp.int32).reshape(N)
    pos1_f = pos1.astype(jnp.int32).reshape(N)
    pos2_f = pos2.astype(jnp.int32).reshape(N)
    out = _emb(word_f, pos1_f, pos2_f, word_table, pos1_table, pos2_table)
    return out.reshape(B, L, OUT_DIM)
